# Initial kernel scaffold; baseline (speedup 1.0000x reference)
import jax
import jax.numpy as jnp
from jax.experimental import pallas as pl


def kernel(population, diffusion_coef, lap_values, src, dst, node_to_city):
    raise NotImplementedError("write your pallas kernel here")


# speedup vs baseline: 4.3983x; 4.3983x over previous
import functools

import jax
import jax.numpy as jnp
from jax import lax
from jax.experimental import pallas as pl
from jax.experimental.pallas import tpu as pltpu
from jax.experimental.pallas import tpu_sc as plsc

N_NODES = 10000
N_EDGES = 320000
N_CITIES = 100
N_ETH = 128

NC = 2
NS = 16
L = 16
NW = NC * NS

K = 128
EDGES_PER_W = -(-N_EDGES // NW)
CHUNKS = -(-EDGES_PER_W // K)
EDGES_PAD = NW * CHUNKS * K
ROWS_PAD = ((N_NODES + NS * K - 1) // (NS * K)) * NS * K
ROWS_PER_TILE = ROWS_PAD // NS
ROW_CHUNKS = ROWS_PER_TILE // K
CB = 8


def _sc_body(pop, coef, n2c, src3, dst3, lap2,
             out,
             acc, lap_v, rows, abuf, cidx, sidx, didx,
             sem_g, sem_c):
    c = lax.axis_index("c")
    s = lax.axis_index("s")
    w = c * NS + s

    pltpu.sync_copy(lap2.at[w], lap_v)

    zvec = jnp.zeros((L,), jnp.float32)

    def zero_row(r, _):
        for b in range(CB):
            abuf[r, pl.ds(b * L, L)] = zvec
        return 0

    lax.fori_loop(0, K, zero_row, 0)
    for j in range(ROW_CHUNKS):
        pltpu.sync_copy(abuf, acc.at[pl.ds(s * ROWS_PER_TILE + j * K, K)])
    plsc.subcore_barrier()

    def chunk_body(i, _):
        pltpu.sync_copy(dst3.at[w, i], didx)
        pltpu.async_copy(pop.at[didx], rows, sem_g)
        pltpu.sync_copy(src3.at[w, i], sidx)
        pltpu.make_async_copy(pop.at[didx], rows, sem_g).wait()

        def scale_body(g, _):
            lv16 = lap_v[pl.ds(i * K + g * L, L)]
            for u in range(L):
                e = g * L + u
                lv = lv16[u]
                for b in range(CB):
                    sl = (e, pl.ds(b * L, L))
                    rows[sl] = rows[sl] * lv
            return 0

        lax.fori_loop(0, K // L, scale_body, 0)
        pltpu.sync_copy(rows, acc.at[sidx], add=True)
        return 0

    lax.fori_loop(0, CHUNKS, chunk_body, 0)
    plsc.subcore_barrier()

    def row_chunk(j, _):
        r0 = s * ROWS_PER_TILE + j * K
        pltpu.sync_copy(n2c.at[pl.ds(r0, K)], cidx)
        pltpu.async_copy(coef.at[cidx], rows, sem_c)
        pltpu.sync_copy(acc.at[pl.ds(r0, K)], abuf)
        pltpu.make_async_copy(coef.at[cidx], rows, sem_c).wait()

        def mul_body(r4, _):
            for u in range(4):
                r = r4 * 4 + u
                for b in range(CB):
                    sl = (r, pl.ds(b * L, L))
                    abuf[sl] = abuf[sl] * rows[sl]
            return 0

        lax.fori_loop(0, K // 4, mul_body, 0)
        pltpu.sync_copy(abuf, out.at[pl.ds(c * ROWS_PAD + r0, K)])
        return 0

    lax.fori_loop(0, ROW_CHUNKS, row_chunk, 0)


def _add_body(a_ref, b_ref, o_ref):
    o_ref[...] = a_ref[...] + b_ref[...]


@jax.jit
def _run(pop, coef, n2c, src3, dst3, lap2):
    f32 = jnp.float32
    kern = pl.kernel(
        _sc_body,
        out_type=jax.ShapeDtypeStruct((NC * ROWS_PAD, N_ETH), f32),
        mesh=plsc.VectorSubcoreMesh(
            core_axis_name="c", subcore_axis_name="s",
            num_cores=NC, num_subcores=NS,
        ),
        scratch_types=[
            pltpu.VMEM_SHARED((ROWS_PAD, N_ETH), f32),
            pltpu.VMEM((CHUNKS * K,), f32),
            pltpu.VMEM((K, N_ETH), f32),
            pltpu.VMEM((K, N_ETH), f32),
            pltpu.VMEM((K,), jnp.int32),
            pltpu.VMEM((K,), jnp.int32),
            pltpu.VMEM((K,), jnp.int32),
            pltpu.SemaphoreType.DMA,
            pltpu.SemaphoreType.DMA,
        ],
    )
    partial = kern(pop, coef, n2c, src3, dst3, lap2)

    BR = 512
    final = pl.pallas_call(
        _add_body,
        out_shape=jax.ShapeDtypeStruct((ROWS_PAD, N_ETH), f32),
        grid=(ROWS_PAD // BR,),
        in_specs=[
            pl.BlockSpec((BR, N_ETH), lambda i: (i, 0)),
            pl.BlockSpec((BR, N_ETH), lambda i: (i + ROWS_PAD // BR, 0)),
        ],
        out_specs=pl.BlockSpec((BR, N_ETH), lambda i: (i, 0)),
    )(partial, partial)
    return final


def kernel(population, diffusion_coef, lap_values, src, dst, node_to_city):
    n2c = jnp.pad(node_to_city, (0, ROWS_PAD - N_NODES))
    pad_e = EDGES_PAD - N_EDGES
    src_p = jnp.pad(src, (0, pad_e), constant_values=N_NODES)
    dst_p = jnp.pad(dst, (0, pad_e))
    lap_p = jnp.pad(lap_values, (0, pad_e))
    src3 = src_p.reshape(NW, CHUNKS, K)
    dst3 = dst_p.reshape(NW, CHUNKS, K)
    lap2 = lap_p.reshape(NW, CHUNKS * K)
    final = _run(population, diffusion_coef, n2c, src3, dst3, lap2)
    return final[:N_NODES]
